# baseline (speedup 1.0000x reference)
import jax
import jax.numpy as jnp
from jax import lax
from jax.experimental import pallas as pl
from jax.experimental.pallas import tpu as pltpu
from jax.experimental.pallas import tpu_sc as plsc

OC = 96
INC = 96
KK = 9
G = 4
PER = 16
SEG = INC * KK
HO = 56
WO = 56
W_IN = 224
RB = 28
NR = HO // RB
_PH = (0, 1, 3)


_NC = 2
_NS = 16
_NW = _NC * _NS
_ROWS_PW = OC // _NW


def _w2_sc_kernel(ci_hbm, means_hbm, out_hbm, idx_v, mrow_v, row_v):
    wid = lax.axis_index("s") * _NC + lax.axis_index("c")
    for t in range(_ROWS_PW):
        oc = wid * _ROWS_PW + t
        pltpu.sync_copy(ci_hbm.at[oc], idx_v)
        pltpu.sync_copy(means_hbm.at[oc], mrow_v)
        for k in range(SEG // PER):
            row_v[pl.ds(k * PER, PER)] = jnp.zeros((PER,), jnp.float32)
        nine = jnp.full((PER,), KK, jnp.int32)
        ninety6 = jnp.full((PER,), INC, jnp.int32)
        for g in range(G):
            v = idx_v[pl.ds(g * PER, PER)]
            kc = lax.rem(v, nine) * ninety6 + lax.div(v, nine)
            mv = mrow_v[pl.ds(g * PER, PER)]
            plsc.store_scatter(row_v, [kc], mv)
        pltpu.sync_copy(row_v, out_hbm.at[oc])


def _build_w2_sc(ci, means):
    return pl.kernel(
        _w2_sc_kernel,
        out_type=jax.ShapeDtypeStruct((OC, SEG), jnp.float32),
        mesh=plsc.VectorSubcoreMesh(core_axis_name="c", subcore_axis_name="s"),
        scratch_types=[pltpu.VMEM((G * PER,), jnp.int32),
                       pltpu.VMEM((G * PER,), jnp.float32),
                       pltpu.VMEM((SEG,), jnp.float32)],
        compiler_params=pltpu.CompilerParams(needs_layout_passes=False),
    )(ci, jnp.repeat(means, PER, axis=1))


def _main_kernel(w2_ref, bias_ref, xv_ref,
                 out_ref, xs_ref, s3_ref, carry_ref, sem_ref):
    b = pl.program_id(0)
    r = pl.program_id(1)
    nb = pl.num_programs(0)
    step = b * NR + r
    slot = jax.lax.rem(step, 2)

    def row_copy(slot_i, bb, rr, i, g):
        return pltpu.make_async_copy(
            xv_ref.at[bb, :, 4 * (rr * RB + g) + _PH[i], :],
            xs_ref.at[slot_i, i, g], sem_ref.at[slot_i, i])

    @pl.when(step == 0)
    def _first_copies():
        for i in range(3):
            for g in range(RB):
                row_copy(0, b, r, i, g).start()

    @pl.when(step + 1 < nb * NR)
    def _next_copies():
        r2 = jax.lax.rem(r + 1, NR)
        b2 = b + jnp.where(r + 1 == NR, 1, 0)
        for i in range(3):
            for g in range(RB):
                row_copy(1 - slot, b2, r2, i, g).start()

    @pl.when(jnp.logical_and(b == 0, r == 0))
    def _build_tables():
        iw = jax.lax.broadcasted_iota(jnp.int32, (W_IN, 3 * WO), 0)
        im = jax.lax.broadcasted_iota(jnp.int32, (W_IN, 3 * WO), 1)
        s3_ref[...] = (iw == 4 * (im % WO) + im // WO - 1).astype(jnp.bfloat16)

    for i in range(3):
        for g in range(RB):
            row_copy(slot, b, r, i, g).wait()

    w2 = w2_ref[...]
    s3 = s3_ref[...]

    pall = xs_ref[slot].astype(jnp.bfloat16)
    qall = jax.lax.dot_general(
        pall.reshape(3 * RB * INC, W_IN), s3, (((1,), (0,)), ((), ())),
        preferred_element_type=jnp.float32).reshape(3, RB, INC, 3 * WO)

    def row_terms(i):
        q = qall[_rt_idx(i)]
        tot = jnp.zeros((RB, WO, OC), jnp.float32)
        for j in range(3):
            wk = w2[:, (3 * i + j) * INC:(3 * i + j + 1) * INC]
            tot = tot + jax.lax.dot_general(
                q[:, :, j * WO:(j + 1) * WO], wk, (((1,), (1,)), ((), ())),
                preferred_element_type=jnp.float32)
        return tot

    acc = row_terms(1) + row_terms(2)
    s0 = row_terms(0)

    carry_in = jnp.where(r > 0, carry_ref[...], 0.0)
    top = acc[:1] + carry_in
    rest = acc[1:] + s0[:RB - 1]
    carry_ref[...] = s0[RB - 1:]

    out_ref[0] = (jnp.concatenate([top, rest], axis=0)
                  + bias_ref[...][None])


def _rt_idx(i):
    return {1: 0, 2: 1, 0: 2}[i]


@jax.jit
def kernel(x, means, bias, col_idx, dest):
    del dest
    B = x.shape[0]
    ci = col_idx.reshape(OC, G * PER)
    bias2 = bias.reshape(1, OC)
    w2 = _build_w2_sc(ci, means)

    return pl.pallas_call(
        _main_kernel,
        grid=(B, NR),
        in_specs=[
            pl.BlockSpec((OC, SEG), lambda b, r: (0, 0)),
            pl.BlockSpec((1, OC), lambda b, r: (0, 0)),
            pl.BlockSpec(memory_space=pl.ANY),
        ],
        out_specs=pl.BlockSpec((1, RB, WO, OC), lambda b, r: (b, r, 0, 0)),
        out_shape=jax.ShapeDtypeStruct((B, HO, WO, OC), jnp.float32),
        scratch_shapes=[pltpu.VMEM((2, 3, RB, INC, W_IN), jnp.float32),
                        pltpu.VMEM((W_IN, 3 * WO), jnp.bfloat16),
                        pltpu.VMEM((1, WO, OC), jnp.float32),
                        pltpu.SemaphoreType.DMA((2, 3))],
        compiler_params=pltpu.CompilerParams(
            dimension_semantics=("arbitrary", "arbitrary")),
    )(w2, bias2, x).transpose(0, 3, 1, 2)
